# baseline (speedup 1.0000x reference)
import functools

import jax
import jax.numpy as jnp
from jax import lax
from jax.experimental import pallas as pl
from jax.experimental.pallas import tpu as pltpu
from jax.experimental.pallas import tpu_sc as plsc

N_NODES = 10000
NPAD = 10240
N_EDGES = 320000
NB = 160
EB = 64
EP = 32 * NB * EB
TRASH = 10000
ROWS_PER_SUB = NPAD // 16

_mesh = plsc.VectorSubcoreMesh(core_axis_name="c", subcore_axis_name="s")


@functools.partial(
    pl.kernel,
    out_type=jax.ShapeDtypeStruct((2, NPAD, 128), jnp.float32),
    mesh=_mesh,
    scratch_types=[
        pltpu.VMEM((NB, EB), jnp.int32),
        pltpu.VMEM_SHARED((NPAD, 128), jnp.float32),
    ] + [pltpu.VMEM((EB, 128), jnp.float32) for _ in range(3)]
      + [pltpu.VMEM((EB,), jnp.int32) for _ in range(6)]
      + [pltpu.SemaphoreType.DMA for _ in range(12)],
)
def _sc_aggregate(table_hbm, src_hbm, dst_hbm, zeros_hbm, out_hbm,
                  src_v, acc, *rest):
    bufs = rest[0:3]
    dsts = rest[3:9]
    gsem = rest[9:12]
    ssem = rest[12:15]
    isem = rest[15:21]
    c = lax.axis_index("c")
    s = lax.axis_index("s")
    wid = s * 2 + c
    pltpu.sync_copy(src_hbm.at[wid], src_v)
    pltpu.sync_copy(zeros_hbm, acc.at[pl.ds(s * ROWS_PER_SUB, ROWS_PER_SUB)])
    plsc.subcore_barrier()

    def start_idx(b, k):
        pltpu.async_copy(dst_hbm.at[wid].at[b], dsts[k], isem[k])

    def wait_idx(k):
        pltpu.make_async_copy(dst_hbm.at[0].at[0], dsts[k], isem[k]).wait()

    def start_gather(b, j):
        pltpu.async_copy(table_hbm.at[src_v.at[b]], bufs[j], gsem[j])

    def wait_gather(j):
        pltpu.make_async_copy(table_hbm.at[pl.ds(0, EB)], bufs[j], gsem[j]).wait()

    def start_scatter(j, k):
        pltpu.async_copy(bufs[j], acc.at[dsts[k]], ssem[j], add=True)

    def wait_scatter(j):
        pltpu.make_async_copy(bufs[j], acc.at[pl.ds(0, EB)], ssem[j]).wait()

    def step(t, with_ws, with_i, with_g):
        j = t % 3
        k = t % 6
        wait_gather(j)
        wait_idx(k)
        start_scatter(j, k)
        if with_ws:
            wait_scatter((t - 1) % 3)
        if with_i:
            start_idx(t + 5, (t - 1) % 6)
        if with_g:
            start_gather(t + 2, (t + 2) % 3)

    for k in range(6):
        start_idx(k, k)
    start_gather(0, 0)
    start_gather(1, 1)
    step(0, False, False, True)

    def outer(i, carry):
        for u in range(6):
            t = 1 + i * 6 + u
            j = (1 + u) % 3
            k = (1 + u) % 6
            wait_gather(j)
            wait_idx(k)
            start_scatter(j, k)
            wait_scatter(u % 3)
            start_idx(t + 5, u % 6)
            start_gather(t + 2, u % 3)
        return carry

    lax.fori_loop(0, (NB - 10) // 6, outer, 0)
    for t in range(NB - 9, NB - 5):
        step(t, True, True, True)
    for t in range(NB - 5, NB - 2):
        step(t, True, False, True)
    for t in range(NB - 2, NB):
        step(t, True, False, False)
    wait_scatter((NB - 1) % 3)

    plsc.subcore_barrier()
    pltpu.sync_copy(
        acc.at[pl.ds(s * ROWS_PER_SUB, ROWS_PER_SUB)],
        out_hbm.at[c].at[pl.ds(s * ROWS_PER_SUB, ROWS_PER_SUB)],
    )


def _mm_body(x_ref, w_ref, o_ref):
    o_ref[...] = jax.lax.dot_general(
        x_ref[...], w_ref[...], (((1,), (0,)), ((), ())),
        preferred_element_type=jnp.float32,
        precision=jax.lax.Precision.HIGHEST,
    )


def _matmul(x, w):
    m, k = x.shape
    _, n = w.shape
    bm = 2000
    n_pad = ((n + 127) // 128) * 128
    if n_pad != n:
        w = jnp.pad(w, ((0, 0), (0, n_pad - n)))
    return pl.pallas_call(
        _mm_body,
        grid=(m // bm,),
        in_specs=[
            pl.BlockSpec((bm, k), lambda i: (i, 0)),
            pl.BlockSpec((k, n_pad), lambda i: (0, 0)),
        ],
        out_specs=pl.BlockSpec((bm, n_pad), lambda i: (i, 0)),
        out_shape=jax.ShapeDtypeStruct((m, n_pad), jnp.float32),
    )(x, w)


def kernel(x, edge_index, W1, b1, W2, b2, W3, b3, W4, b4, W5, b5, W6, b6):
    src = edge_index[0].astype(jnp.int32)
    dst = edge_index[1].astype(jnp.int32)
    pad = EP - N_EDGES
    src_p = jnp.concatenate([src, jnp.zeros((pad,), jnp.int32)]).reshape(32, NB, EB)
    dst_p = jnp.concatenate([dst, jnp.full((pad,), TRASH, jnp.int32)]).reshape(32, NB, EB)

    zeros128 = jnp.zeros((ROWS_PER_SUB, 128), jnp.float32)

    ones_tab = jnp.ones((N_NODES, 128), jnp.float32)
    dpart = _sc_aggregate(ones_tab, src_p, dst_p, zeros128)
    deg = dpart[0, :N_NODES, 0] + dpart[1, :N_NODES, 0] + 1.0
    dinv = lax.rsqrt(jnp.maximum(deg, 1e-12))[:, None]

    def aggregate(hs):
        f = hs.shape[1]
        outs = []
        for ci in range(f // 128):
            part = _sc_aggregate(hs[:, ci * 128:(ci + 1) * 128], src_p, dst_p, zeros128)
            outs.append(part[0, :N_NODES] + part[1, :N_NODES])
        return outs[0] if len(outs) == 1 else jnp.concatenate(outs, axis=1)

    xs = dinv * x
    u = dinv * (aggregate(xs) + xs)
    h = jax.nn.relu(_matmul(u, W1)[:, :640] + b1)

    for W, b, act in ((W2, b2, True), (W3, b3, True), (W4, b4, True),
                      (W5, b5, True), (W6, b6, False)):
        n_out = W.shape[1]
        t = _matmul(h, W)
        hs = dinv * t
        h = dinv * (aggregate(hs) + hs)[:, :n_out] + b
        if act:
            h = jax.nn.relu(h)
    return jax.nn.log_softmax(h, axis=1)


# speedup vs baseline: 1.0007x; 1.0007x over previous
import functools

import jax
import jax.numpy as jnp
from jax import lax
from jax.experimental import pallas as pl
from jax.experimental.pallas import tpu as pltpu
from jax.experimental.pallas import tpu_sc as plsc

N_NODES = 10000
NPAD = 10240
N_EDGES = 320000
NB = 80
EB = 128
EP = 32 * NB * EB
TRASH = 10000
ROWS_PER_SUB = NPAD // 16
BM = 2000

_mesh = plsc.VectorSubcoreMesh(core_axis_name="c", subcore_axis_name="s")


@functools.partial(
    pl.kernel,
    out_type=jax.ShapeDtypeStruct((2, NPAD, 128), jnp.float32),
    mesh=_mesh,
    scratch_types=[
        pltpu.VMEM((NB, EB), jnp.int32),
        pltpu.VMEM((NB, EB), jnp.int32),
        pltpu.VMEM((EB, 128), jnp.float32),
        pltpu.VMEM_SHARED((NPAD, 128), jnp.float32),
        pltpu.SemaphoreType.DMA,
    ],
)
def _sc_aggregate(table_hbm, src_hbm, dst_hbm, zeros_hbm, out_hbm,
                  src_v, dst_v, buf, acc, sem):
    c = lax.axis_index("c")
    s = lax.axis_index("s")
    wid = s * 2 + c
    pltpu.sync_copy(src_hbm.at[wid], src_v)
    pltpu.sync_copy(dst_hbm.at[wid], dst_v)
    pltpu.sync_copy(zeros_hbm, acc.at[pl.ds(s * ROWS_PER_SUB, ROWS_PER_SUB)])
    plsc.subcore_barrier()

    def body(b, carry):
        pltpu.async_copy(table_hbm.at[src_v.at[b]], buf, sem).wait()
        pltpu.sync_copy(buf, acc.at[dst_v.at[b]], add=True)
        return carry

    lax.fori_loop(0, NB, body, 0)
    plsc.subcore_barrier()
    pltpu.sync_copy(
        acc.at[pl.ds(s * ROWS_PER_SUB, ROWS_PER_SUB)],
        out_hbm.at[c].at[pl.ds(s * ROWS_PER_SUB, ROWS_PER_SUB)],
    )


def _dot(a, b):
    return jax.lax.dot_general(
        a, b, (((1,), (0,)), ((), ())),
        preferred_element_type=jnp.float32,
        precision=jax.lax.Precision.HIGHEST,
    )


def _first_body(aggx_ref, xs_ref, dinv_ref, w1_ref, b1_ref, w2_ref, o_ref):
    u = dinv_ref[...] * (aggx_ref[...] + xs_ref[...])
    h = jnp.maximum(_dot(u, w1_ref[...]) + b1_ref[...], 0.0)
    o_ref[...] = dinv_ref[...] * _dot(h, w2_ref[...])


def _mid_body(agg_ref, hs_ref, dinv_ref, b_ref, w_ref, o_ref):
    h = jnp.maximum(dinv_ref[...] * (agg_ref[...] + hs_ref[...]) + b_ref[...], 0.0)
    o_ref[...] = dinv_ref[...] * _dot(h, w_ref[...])


def _last_body(agg_ref, hs_ref, dinv_ref, b_ref, o_ref):
    logits = dinv_ref[...] * (agg_ref[...] + hs_ref[...]) + b_ref[...]
    mask = jax.lax.broadcasted_iota(jnp.int32, logits.shape, 1) < 10
    logits = jnp.where(mask, logits, -jnp.inf)
    m = jnp.max(logits, axis=1, keepdims=True)
    lse = jnp.log(jnp.sum(jnp.where(mask, jnp.exp(logits - m), 0.0),
                          axis=1, keepdims=True))
    o_ref[...] = logits - m - lse


def _row_spec(f):
    return pl.BlockSpec((BM, f), lambda i: (i, 0))


def _full_spec(shape):
    return pl.BlockSpec(shape, lambda i: (0, 0))


def _first_tc(aggx, xs, dinv, w1, b1, w2):
    f1, f2 = w1.shape[1], w2.shape[1]
    return pl.pallas_call(
        _first_body,
        grid=(N_NODES // BM,),
        in_specs=[_row_spec(128), _row_spec(128), _row_spec(1),
                  _full_spec((128, f1)), _full_spec((1, f1)),
                  _full_spec((f1, f2))],
        out_specs=_row_spec(f2),
        out_shape=jax.ShapeDtypeStruct((N_NODES, f2), jnp.float32),
    )(aggx, xs, dinv, w1, b1.reshape(1, f1), w2)


def _mid_tc(agg, hs, dinv, b, w):
    f, fn = w.shape
    return pl.pallas_call(
        _mid_body,
        grid=(N_NODES // BM,),
        in_specs=[_row_spec(f), _row_spec(f), _row_spec(1),
                  _full_spec((1, f)), _full_spec((f, fn))],
        out_specs=_row_spec(fn),
        out_shape=jax.ShapeDtypeStruct((N_NODES, fn), jnp.float32),
    )(agg, hs, dinv, b.reshape(1, f), w)


def _last_tc(agg, hs, dinv, b):
    return pl.pallas_call(
        _last_body,
        grid=(N_NODES // BM,),
        in_specs=[_row_spec(128), _row_spec(128), _row_spec(1),
                  _full_spec((1, 128))],
        out_specs=_row_spec(128),
        out_shape=jax.ShapeDtypeStruct((N_NODES, 128), jnp.float32),
    )(agg, hs, dinv, b)


def kernel(x, edge_index, W1, b1, W2, b2, W3, b3, W4, b4, W5, b5, W6, b6):
    src = edge_index[0].astype(jnp.int32)
    dst = edge_index[1].astype(jnp.int32)
    pad = EP - N_EDGES
    src_p = jnp.concatenate([src, jnp.zeros((pad,), jnp.int32)]).reshape(32, NB, EB)
    dst_p = jnp.concatenate([dst, jnp.full((pad,), TRASH, jnp.int32)]).reshape(32, NB, EB)

    zeros128 = jnp.zeros((ROWS_PER_SUB, 128), jnp.float32)

    def aggregate(hs):
        f = hs.shape[1]
        outs = []
        for ci in range(f // 128):
            part = _sc_aggregate(hs[:, ci * 128:(ci + 1) * 128], src_p, dst_p, zeros128)
            outs.append(part[0, :N_NODES] + part[1, :N_NODES])
        return outs[0] if len(outs) == 1 else jnp.concatenate(outs, axis=1)

    ones_tab = jnp.ones((N_NODES, 128), jnp.float32)
    dpart = _sc_aggregate(ones_tab, src_p, dst_p, zeros128)
    deg = dpart[0, :N_NODES, 0] + dpart[1, :N_NODES, 0] + 1.0
    dinv = lax.rsqrt(jnp.maximum(deg, 1e-12))[:, None]

    xs = dinv * x
    hs = _first_tc(aggregate(xs), xs, dinv, W1, b1, W2)
    for W, b in ((W3, b2), (W4, b3), (W5, b4)):
        hs = _mid_tc(aggregate(hs), hs, dinv, b, W)
    W6p = jnp.pad(W6, ((0, 0), (0, 118)))
    b6p = jnp.pad(b6, (0, 118)).reshape(1, 128)
    hs6 = _mid_tc(aggregate(hs), hs, dinv, b5, W6p)
    out = _last_tc(aggregate(hs6), hs6, dinv, b6p)
    return out[:, :10]


# speedup vs baseline: 1.0017x; 1.0010x over previous
import functools

import jax
import jax.numpy as jnp
from jax import lax
from jax.experimental import pallas as pl
from jax.experimental.pallas import tpu as pltpu
from jax.experimental.pallas import tpu_sc as plsc

N_NODES = 10000
NPAD = 10240
N_EDGES = 320000
NB = 80
EB = 128
EP = 32 * NB * EB
TRASH = 10000
ROWS_PER_SUB = NPAD // 16
BM = 2000

_mesh = plsc.VectorSubcoreMesh(core_axis_name="c", subcore_axis_name="s")


@functools.partial(
    pl.kernel,
    out_type=jax.ShapeDtypeStruct((2, NPAD, 128), jnp.float32),
    mesh=_mesh,
    scratch_types=[
        pltpu.VMEM((NB, EB), jnp.int32),
        pltpu.VMEM((NB, EB), jnp.int32),
        pltpu.VMEM((EB, 128), jnp.float32),
        pltpu.VMEM_SHARED((NPAD, 128), jnp.float32),
        pltpu.SemaphoreType.DMA,
    ],
)
def _sc_aggregate(table_hbm, src_hbm, dst_hbm, zeros_hbm, out_hbm,
                  src_v, dst_v, buf, acc, sem):
    c = lax.axis_index("c")
    s = lax.axis_index("s")
    wid = s * 2 + c
    pltpu.sync_copy(src_hbm.at[wid], src_v)
    pltpu.sync_copy(dst_hbm.at[wid], dst_v)
    pltpu.sync_copy(zeros_hbm, acc.at[pl.ds(s * ROWS_PER_SUB, ROWS_PER_SUB)])
    plsc.subcore_barrier()

    def body(b, carry):
        pltpu.async_copy(table_hbm.at[src_v.at[b]], buf, sem).wait()
        pltpu.sync_copy(buf, acc.at[dst_v.at[b]], add=True)
        return carry

    lax.fori_loop(0, NB, body, 0)
    plsc.subcore_barrier()
    pltpu.sync_copy(
        acc.at[pl.ds(s * ROWS_PER_SUB, ROWS_PER_SUB)],
        out_hbm.at[c].at[pl.ds(s * ROWS_PER_SUB, ROWS_PER_SUB)],
    )


def _dot(a, b):
    return jax.lax.dot_general(
        a, b, (((1,), (0,)), ((), ())),
        preferred_element_type=jnp.float32,
        precision=jax.lax.Precision.HIGHEST,
    )


def _first_body(aggx_ref, xs_ref, dinv_ref, w1_ref, b1_ref, w2_ref, o_ref):
    u = dinv_ref[...] * (aggx_ref[...] + xs_ref[...])
    h = jnp.maximum(_dot(u, w1_ref[...]) + b1_ref[...], 0.0)
    o_ref[...] = dinv_ref[...] * _dot(h, w2_ref[...])


def _mid_body(agg_ref, hs_ref, dinv_ref, b_ref, w_ref, o_ref):
    h = jnp.maximum(dinv_ref[...] * (agg_ref[...] + hs_ref[...]) + b_ref[...], 0.0)
    o_ref[...] = dinv_ref[...] * _dot(h, w_ref[...])


def _last_body(agg_ref, hs_ref, dinv_ref, b_ref, o_ref):
    logits = dinv_ref[...] * (agg_ref[...] + hs_ref[...]) + b_ref[...]
    mask = jax.lax.broadcasted_iota(jnp.int32, logits.shape, 1) < 10
    logits = jnp.where(mask, logits, -jnp.inf)
    m = jnp.max(logits, axis=1, keepdims=True)
    lse = jnp.log(jnp.sum(jnp.where(mask, jnp.exp(logits - m), 0.0),
                          axis=1, keepdims=True))
    o_ref[...] = logits - m - lse


def _row_spec(f):
    return pl.BlockSpec((BM, f), lambda i: (i, 0))


def _full_spec(shape):
    return pl.BlockSpec(shape, lambda i: (0, 0))


def _first_tc(aggx, xs, dinv, w1, b1, w2):
    f1, f2 = w1.shape[1], w2.shape[1]
    return pl.pallas_call(
        _first_body,
        grid=(N_NODES // BM,),
        in_specs=[_row_spec(128), _row_spec(128), _row_spec(1),
                  _full_spec((128, f1)), _full_spec((1, f1)),
                  _full_spec((f1, f2))],
        out_specs=_row_spec(f2),
        out_shape=jax.ShapeDtypeStruct((N_NODES, f2), jnp.float32),
    )(aggx, xs, dinv, w1, b1.reshape(1, f1), w2)


def _mid_tc(agg, hs, dinv, b, w):
    f, fn = w.shape
    return pl.pallas_call(
        _mid_body,
        grid=(N_NODES // BM,),
        in_specs=[_row_spec(f), _row_spec(f), _row_spec(1),
                  _full_spec((1, f)), _full_spec((f, fn))],
        out_specs=_row_spec(fn),
        out_shape=jax.ShapeDtypeStruct((N_NODES, fn), jnp.float32),
    )(agg, hs, dinv, b.reshape(1, f), w)


def _last_tc(agg, hs, dinv, b):
    return pl.pallas_call(
        _last_body,
        grid=(N_NODES // BM,),
        in_specs=[_row_spec(128), _row_spec(128), _row_spec(1),
                  _full_spec((1, 128))],
        out_specs=_row_spec(128),
        out_shape=jax.ShapeDtypeStruct((N_NODES, 128), jnp.float32),
    )(agg, hs, dinv, b)


def kernel(x, edge_index, W1, b1, W2, b2, W3, b3, W4, b4, W5, b5, W6, b6):
    src = edge_index[0].astype(jnp.int32)
    dst = edge_index[1].astype(jnp.int32)
    pad = EP - N_EDGES
    src_p = jnp.concatenate([src, jnp.zeros((pad,), jnp.int32)]).reshape(32, NB, EB)
    trash_rows = TRASH + (jnp.arange(pad, dtype=jnp.int32) % (NPAD - N_NODES))
    dst_p = jnp.concatenate([dst, trash_rows]).reshape(32, NB, EB)

    zeros128 = jnp.zeros((ROWS_PER_SUB, 128), jnp.float32)

    def aggregate(hs):
        f = hs.shape[1]
        outs = []
        for ci in range(f // 128):
            part = _sc_aggregate(hs[:, ci * 128:(ci + 1) * 128], src_p, dst_p, zeros128)
            outs.append(part[0, :N_NODES] + part[1, :N_NODES])
        return outs[0] if len(outs) == 1 else jnp.concatenate(outs, axis=1)

    ones_tab = jnp.ones((N_NODES, 128), jnp.float32)
    dpart = _sc_aggregate(ones_tab, src_p, dst_p, zeros128)
    deg = dpart[0, :N_NODES, 0] + dpart[1, :N_NODES, 0] + 1.0
    dinv = lax.rsqrt(jnp.maximum(deg, 1e-12))[:, None]

    xs = dinv * x
    hs = _first_tc(aggregate(xs), xs, dinv, W1, b1, W2)
    for W, b in ((W3, b2), (W4, b3), (W5, b4)):
        hs = _mid_tc(aggregate(hs), hs, dinv, b, W)
    W6p = jnp.pad(W6, ((0, 0), (0, 118)))
    b6p = jnp.pad(b6, (0, 118)).reshape(1, 128)
    hs6 = _mid_tc(aggregate(hs), hs, dinv, b5, W6p)
    out = _last_tc(aggregate(hs6), hs6, dinv, b6p)
    return out[:, :10]


# speedup vs baseline: 1.4854x; 1.4828x over previous
import functools

import jax
import jax.numpy as jnp
from jax import lax
from jax.experimental import pallas as pl
from jax.experimental.pallas import tpu as pltpu
from jax.experimental.pallas import tpu_sc as plsc

N_NODES = 10000
NPAD = 10240
N_EDGES = 320000
NB = 79
EB = 128
EP = 32 * NB * EB
TRASH = 10000
ROWS_PER_SUB = NPAD // 16
BM = 2000

_mesh = plsc.VectorSubcoreMesh(core_axis_name="c", subcore_axis_name="s")


@functools.partial(
    pl.kernel,
    out_type=jax.ShapeDtypeStruct((2, NPAD, 128), jnp.float32),
    mesh=_mesh,
    scratch_types=[
        pltpu.VMEM((NB, EB), jnp.int32),
        pltpu.VMEM((NB, EB), jnp.int32),
        pltpu.VMEM((EB, 128), jnp.float32),
        pltpu.VMEM_SHARED((NPAD, 128), jnp.float32),
        pltpu.SemaphoreType.DMA,
    ],
)
def _sc_aggregate(table_hbm, src_hbm, dst_hbm, zeros_hbm, out_hbm,
                  src_v, dst_v, buf, acc, sem):
    c = lax.axis_index("c")
    s = lax.axis_index("s")
    wid = s * 2 + c
    pltpu.sync_copy(src_hbm.at[wid], src_v)
    pltpu.sync_copy(dst_hbm.at[wid], dst_v)
    pltpu.sync_copy(zeros_hbm, acc.at[pl.ds(s * ROWS_PER_SUB, ROWS_PER_SUB)])
    plsc.subcore_barrier()

    def body(b, carry):
        pltpu.async_copy(table_hbm.at[src_v.at[b]], buf, sem).wait()
        pltpu.sync_copy(buf, acc.at[dst_v.at[b]], add=True)
        return carry

    lax.fori_loop(0, NB, body, 0)
    plsc.subcore_barrier()
    pltpu.sync_copy(
        acc.at[pl.ds(s * ROWS_PER_SUB, ROWS_PER_SUB)],
        out_hbm.at[c].at[pl.ds(s * ROWS_PER_SUB, ROWS_PER_SUB)],
    )


def _dot(a, b):
    return jax.lax.dot_general(
        a, b, (((1,), (0,)), ((), ())),
        preferred_element_type=jnp.float32,
        precision=jax.lax.Precision.HIGHEST,
    )


def _first_body(aggx_ref, xs_ref, dinv_ref, w1_ref, b1_ref, w2_ref, o_ref):
    u = dinv_ref[...] * (aggx_ref[...] + xs_ref[...])
    h = jnp.maximum(_dot(u, w1_ref[...]) + b1_ref[...], 0.0)
    o_ref[...] = dinv_ref[...] * _dot(h, w2_ref[...])


def _mid_body(agg_ref, hs_ref, dinv_ref, b_ref, w_ref, o_ref):
    h = jnp.maximum(dinv_ref[...] * (agg_ref[...] + hs_ref[...]) + b_ref[...], 0.0)
    o_ref[...] = dinv_ref[...] * _dot(h, w_ref[...])


def _last_body(agg_ref, hs_ref, dinv_ref, b_ref, o_ref):
    logits = dinv_ref[...] * (agg_ref[...] + hs_ref[...]) + b_ref[...]
    mask = jax.lax.broadcasted_iota(jnp.int32, logits.shape, 1) < 10
    logits = jnp.where(mask, logits, -jnp.inf)
    m = jnp.max(logits, axis=1, keepdims=True)
    lse = jnp.log(jnp.sum(jnp.where(mask, jnp.exp(logits - m), 0.0),
                          axis=1, keepdims=True))
    o_ref[...] = logits - m - lse


def _row_spec(f):
    return pl.BlockSpec((BM, f), lambda i: (i, 0))


def _full_spec(shape):
    return pl.BlockSpec(shape, lambda i: (0, 0))


def _first_tc(aggx, xs, dinv, w1, b1, w2):
    f1, f2 = w1.shape[1], w2.shape[1]
    return pl.pallas_call(
        _first_body,
        grid=(N_NODES // BM,),
        in_specs=[_row_spec(128), _row_spec(128), _row_spec(1),
                  _full_spec((128, f1)), _full_spec((1, f1)),
                  _full_spec((f1, f2))],
        out_specs=_row_spec(f2),
        out_shape=jax.ShapeDtypeStruct((N_NODES, f2), jnp.float32),
    )(aggx, xs, dinv, w1, b1.reshape(1, f1), w2)


def _mid_tc(agg, hs, dinv, b, w):
    f, fn = w.shape
    return pl.pallas_call(
        _mid_body,
        grid=(N_NODES // BM,),
        in_specs=[_row_spec(f), _row_spec(f), _row_spec(1),
                  _full_spec((1, f)), _full_spec((f, fn))],
        out_specs=_row_spec(fn),
        out_shape=jax.ShapeDtypeStruct((N_NODES, fn), jnp.float32),
    )(agg, hs, dinv, b.reshape(1, f), w)


def _last_tc(agg, hs, dinv, b):
    return pl.pallas_call(
        _last_body,
        grid=(N_NODES // BM,),
        in_specs=[_row_spec(128), _row_spec(128), _row_spec(1),
                  _full_spec((1, 128))],
        out_specs=_row_spec(128),
        out_shape=jax.ShapeDtypeStruct((N_NODES, 128), jnp.float32),
    )(agg, hs, dinv, b)


def kernel(x, edge_index, W1, b1, W2, b2, W3, b3, W4, b4, W5, b5, W6, b6):
    src = edge_index[0].astype(jnp.int32)
    dst = edge_index[1].astype(jnp.int32)
    pad = EP - N_EDGES
    src_p = jnp.concatenate([src, jnp.zeros((pad,), jnp.int32)]).reshape(32, NB, EB)
    trash_rows = TRASH + (jnp.arange(pad, dtype=jnp.int32) % (NPAD - N_NODES))
    dst_p = jnp.concatenate([dst, trash_rows]).reshape(32, NB, EB)

    zeros128 = jnp.zeros((ROWS_PER_SUB, 128), jnp.float32)

    def aggregate(hs):
        f = hs.shape[1]
        outs = []
        for ci in range(f // 128):
            part = _sc_aggregate(hs[:, ci * 128:(ci + 1) * 128], src_p, dst_p, zeros128)
            outs.append(part[0, :N_NODES] + part[1, :N_NODES])
        return outs[0] if len(outs) == 1 else jnp.concatenate(outs, axis=1)

    ones_tab = jnp.ones((N_NODES, 128), jnp.float32)
    dpart = _sc_aggregate(ones_tab, src_p, dst_p, zeros128)
    deg = dpart[0, :N_NODES, 0] + dpart[1, :N_NODES, 0] + 1.0
    dinv = lax.rsqrt(jnp.maximum(deg, 1e-12))[:, None]

    xs = dinv * x
    hs = _first_tc(aggregate(xs), xs, dinv, W1, b1, W2)
    for W, b in ((W3, b2), (W4, b3), (W5, b4)):
        hs = _mid_tc(aggregate(hs), hs, dinv, b, W)
    W6p = jnp.pad(W6, ((0, 0), (0, 118)))
    b6p = jnp.pad(b6, (0, 118)).reshape(1, 128)
    hs6 = _mid_tc(aggregate(hs), hs, dinv, b5, W6p)
    out = _last_tc(aggregate(hs6), hs6, dinv, b6p)
    return out[:, :10]
